# Initial kernel scaffold; baseline (speedup 1.0000x reference)
import jax
import jax.numpy as jnp
from jax.experimental import pallas as pl


def kernel(x, edge_index, W1, b1, W2, b2, W3, b3, W4, b4):
    raise NotImplementedError("write your pallas kernel here")


# speedup vs baseline: 6.2845x; 6.2845x over previous
import functools
import jax
import jax.numpy as jnp
from jax import lax
from jax.experimental import pallas as pl
from jax.experimental.pallas import tpu as pltpu
from jax.experimental.pallas import tpu_sc as plsc

_N = 10000
_D = 128
_E = 320000
_NC = 2
_NS = 16
_NW = _NC * _NS
_CH = 128
_NCHUNK = _E // _CH
_CPW = -(-_NCHUNK // _NW)
_RPT = 640
_NPAD = _NS * _RPT

_sc_mesh = plsc.VectorSubcoreMesh(
    core_axis_name="c", subcore_axis_name="s", num_cores=_NC, num_subcores=_NS
)


def _seg_sum_body(x_hbm, src_hbm, dst_hbm, zero_hbm, out_hbm,
                  src_v, dst_v, rows_v, acc_sh, gsem):
    c = lax.axis_index("c")
    s = lax.axis_index("s")
    wid = s * _NC + c

    pltpu.sync_copy(zero_hbm, acc_sh.at[pl.ds(s * _RPT, _RPT)])
    plsc.subcore_barrier()

    def chunk(j, carry):
        cid = j * _NW + wid

        @pl.when(cid < _NCHUNK)
        def _():
            pltpu.sync_copy(src_hbm.at[cid], src_v)
            pltpu.async_copy(x_hbm.at[src_v], rows_v, gsem).wait()
            pltpu.sync_copy(dst_hbm.at[cid], dst_v)
            pltpu.sync_copy(rows_v, acc_sh.at[dst_v], add=True)

        return carry

    lax.fori_loop(0, _CPW, chunk, 0)

    plsc.subcore_barrier()
    pltpu.sync_copy(acc_sh.at[pl.ds(s * _RPT, _RPT)],
                    out_hbm.at[c, pl.ds(s * _RPT, _RPT)])


_seg_sum = pl.kernel(
    _seg_sum_body,
    out_type=jax.ShapeDtypeStruct((_NC, _NPAD, _D), jnp.float32),
    mesh=_sc_mesh,
    scratch_types=[
        pltpu.VMEM((_CH,), jnp.int32),
        pltpu.VMEM((_CH,), jnp.int32),
        pltpu.VMEM((_CH, _D), jnp.float32),
        pltpu.VMEM_SHARED((_NPAD, _D), jnp.float32),
        pltpu.SemaphoreType.DMA,
    ],
)


_ROWS_BLK = 1000


def _mlp1_body(x_ref, p_ref, w1_ref, b1_ref, w2_ref, b2_ref, o_ref):
    h = x_ref[...] + p_ref[0] + p_ref[1]
    a = jnp.dot(h, w1_ref[...], preferred_element_type=jnp.float32) + b1_ref[...]
    a = jnp.maximum(a, 0.0)
    z = jnp.dot(a, w2_ref[...], preferred_element_type=jnp.float32) + b2_ref[...]
    o_ref[...] = jnp.maximum(z, 0.0)


def _mlp2_body(h_ref, q_ref, w3_ref, b3_ref, w4_ref, b4_ref, o_ref):
    g = h_ref[...] + q_ref[0] + q_ref[1]
    a = jnp.dot(g, w3_ref[...], preferred_element_type=jnp.float32) + b3_ref[...]
    a = jnp.maximum(a, 0.0)
    z = jnp.dot(a, w4_ref[...], preferred_element_type=jnp.float32) + b4_ref[...]
    m = jnp.max(z, axis=1, keepdims=True)
    e = z - m
    o_ref[...] = e - jnp.log(jnp.sum(jnp.exp(e), axis=1, keepdims=True))


def _row_blocked_call(body):
    blk = lambda: pl.BlockSpec((_ROWS_BLK, _D), lambda i: (i, 0))
    part = pl.BlockSpec((_NC, _ROWS_BLK, _D), lambda i: (0, i, 0))
    full = lambda: pl.BlockSpec((_D, _D), lambda i: (0, 0))
    bias = lambda: pl.BlockSpec((1, _D), lambda i: (0, 0))
    return pl.pallas_call(
        body,
        grid=(_N // _ROWS_BLK,),
        in_specs=[blk(), part, full(), bias(), full(), bias()],
        out_specs=blk(),
        out_shape=jax.ShapeDtypeStruct((_N, _D), jnp.float32),
    )


_mlp1 = _row_blocked_call(_mlp1_body)
_mlp2 = _row_blocked_call(_mlp2_body)


def kernel(x, edge_index, W1, b1, W2, b2, W3, b3, W4, b4):
    src2d = edge_index[0].reshape(_NCHUNK, _CH)
    dst2d = edge_index[1].reshape(_NCHUNK, _CH)
    zeros = jnp.zeros((_RPT, _D), jnp.float32)

    p = _seg_sum(x, src2d, dst2d, zeros)
    h = _mlp1(x, p, W1, b1.reshape(1, _D), W2, b2.reshape(1, _D))
    q = _seg_sum(h, src2d, dst2d, zeros)
    return _mlp2(h, q, W3, b3.reshape(1, _D), W4, b4.reshape(1, _D))


# speedup vs baseline: 8.0445x; 1.2801x over previous
import jax
import jax.numpy as jnp
from jax import lax
from jax.experimental import pallas as pl
from jax.experimental.pallas import tpu as pltpu
from jax.experimental.pallas import tpu_sc as plsc

_N = 10000
_D = 128
_E = 320000
_NC = 2
_NS = 16
_NH = _N // _NC
_NTRASH = 120
_NACC = _NH + _NTRASH
_RPT = _NACC // _NS
_CH = 128
_K = 160
_EPAD = _NS * _K * _CH
_NBUF = 2

_sc_mesh = plsc.VectorSubcoreMesh(
    core_axis_name="c", subcore_axis_name="s", num_cores=_NC, num_subcores=_NS
)


def _seg_sum_body(x_hbm, src_hbm, dst_hbm, zero_hbm, out_hbm,
                  src_v, dst_v, rows0, rows1, acc_sh,
                  zsem, gsem0, gsem1):
    c = lax.axis_index("c")
    s = lax.axis_index("s")
    rows = [rows0, rows1]
    gsem = [gsem0, gsem1]

    zcp = pltpu.async_copy(zero_hbm, acc_sh.at[pl.ds(s * _RPT, _RPT)], zsem)
    pltpu.sync_copy(src_hbm.at[s], src_v)
    pltpu.sync_copy(dst_hbm.at[c * _NS + s], dst_v)
    for b in range(_NBUF - 1):
        pltpu.async_copy(x_hbm.at[src_v.at[b]], rows[b], gsem[b])
    zcp.wait()
    plsc.subcore_barrier()

    def group(g, carry):
        for b in range(_NBUF):
            j = g * _NBUF + b
            nxt = (b + _NBUF - 1) % _NBUF

            @pl.when(j + _NBUF - 1 < _K)
            def _():
                pltpu.async_copy(x_hbm.at[src_v.at[j + _NBUF - 1]],
                                 rows[nxt], gsem[nxt])

            pltpu.make_async_copy(x_hbm.at[src_v.at[j]], rows[b], gsem[b]).wait()
            pltpu.sync_copy(rows[b], acc_sh.at[dst_v.at[j]], add=True)
        return carry

    lax.fori_loop(0, _K // _NBUF, group, 0)

    plsc.subcore_barrier()
    pltpu.sync_copy(acc_sh.at[pl.ds(s * _RPT, _RPT)],
                    out_hbm.at[c, pl.ds(s * _RPT, _RPT)])


_seg_sum = pl.kernel(
    _seg_sum_body,
    out_type=jax.ShapeDtypeStruct((_NC, _NACC, _D), jnp.float32),
    mesh=_sc_mesh,
    scratch_types=[
        pltpu.VMEM((_K, _CH), jnp.int32),
        pltpu.VMEM((_K, _CH), jnp.int32),
        pltpu.VMEM((_CH, _D), jnp.float32),
        pltpu.VMEM((_CH, _D), jnp.float32),
        pltpu.VMEM_SHARED((_NACC, _D), jnp.float32),
        pltpu.SemaphoreType.DMA,
        pltpu.SemaphoreType.DMA,
        pltpu.SemaphoreType.DMA,
    ],
)


_NCHUNK = _EPAD // _CH
_ECHUNK = _E // _CH


def _edge_prep_body(edge_ref, src_out, dst_out):
    src2d = edge_ref[0].reshape(_ECHUNK, _CH)
    dst2d = edge_ref[1].reshape(_ECHUNK, _CH)
    npad = _NCHUNK - _ECHUNK
    padk = (lax.broadcasted_iota(jnp.int32, (npad, _CH), 0) * _CH
            + lax.broadcasted_iota(jnp.int32, (npad, _CH), 1))
    src_out[...] = jnp.concatenate(
        [src2d, (padk * 131) % _N], axis=0)
    eids = (lax.broadcasted_iota(jnp.int32, (_ECHUNK, _CH), 0) * _CH
            + lax.broadcasted_iota(jnp.int32, (_ECHUNK, _CH), 1))
    trash = _NH + (eids % _NTRASH)
    pad_trash = _NH + ((_E + padk) % _NTRASH)
    cores = []
    for cc in range(_NC):
        local = dst2d - cc * _NH
        own = (local >= 0) & (local < _NH)
        main = jnp.where(own, local, trash)
        cores.append(jnp.concatenate([main, pad_trash], axis=0))
    dst_out[...] = jnp.stack(cores)


_edge_prep = pl.pallas_call(
    _edge_prep_body,
    out_shape=(
        jax.ShapeDtypeStruct((_NCHUNK, _CH), jnp.int32),
        jax.ShapeDtypeStruct((_NC, _NCHUNK, _CH), jnp.int32),
    ),
)


_ROWS_BLK = 1000
_BPC = _NH // _ROWS_BLK


def _mlp1_body(x_ref, p_ref, w1_ref, b1_ref, w2_ref, b2_ref, o_ref):
    h = x_ref[...] + p_ref[0]
    a = jnp.dot(h, w1_ref[...], preferred_element_type=jnp.float32) + b1_ref[...]
    a = jnp.maximum(a, 0.0)
    z = jnp.dot(a, w2_ref[...], preferred_element_type=jnp.float32) + b2_ref[...]
    o_ref[...] = jnp.maximum(z, 0.0)


def _mlp2_body(h_ref, q_ref, w3_ref, b3_ref, w4_ref, b4_ref, o_ref):
    g = h_ref[...] + q_ref[0]
    a = jnp.dot(g, w3_ref[...], preferred_element_type=jnp.float32) + b3_ref[...]
    a = jnp.maximum(a, 0.0)
    z = jnp.dot(a, w4_ref[...], preferred_element_type=jnp.float32) + b4_ref[...]
    m = jnp.max(z, axis=1, keepdims=True)
    e = z - m
    o_ref[...] = e - jnp.log(jnp.sum(jnp.exp(e), axis=1, keepdims=True))


def _row_blocked_call(body):
    blk = lambda: pl.BlockSpec((_ROWS_BLK, _D), lambda i: (i, 0))
    part = pl.BlockSpec((1, _ROWS_BLK, _D), lambda i: (i // _BPC, i % _BPC, 0))
    full = lambda: pl.BlockSpec((_D, _D), lambda i: (0, 0))
    bias = lambda: pl.BlockSpec((1, _D), lambda i: (0, 0))
    return pl.pallas_call(
        body,
        grid=(_N // _ROWS_BLK,),
        in_specs=[blk(), part, full(), bias(), full(), bias()],
        out_specs=blk(),
        out_shape=jax.ShapeDtypeStruct((_N, _D), jnp.float32),
    )


_mlp1 = _row_blocked_call(_mlp1_body)
_mlp2 = _row_blocked_call(_mlp2_body)


def kernel(x, edge_index, W1, b1, W2, b2, W3, b3, W4, b4):
    srcp, dstp = _edge_prep(edge_index)
    srcp = srcp.reshape(_NS, _K, _CH)
    dst3d = dstp.reshape(_NC * _NS, _K, _CH)
    zeros = jnp.zeros((_RPT, _D), jnp.float32)

    p = _seg_sum(x, srcp, dst3d, zeros)
    h = _mlp1(x, p, W1, b1.reshape(1, _D), W2, b2.reshape(1, _D))
    q = _seg_sum(h, srcp, dst3d, zeros)
    return _mlp2(h, q, W3, b3.reshape(1, _D), W4, b4.reshape(1, _D))


# speedup vs baseline: 8.0842x; 1.0049x over previous
import jax
import jax.numpy as jnp
from jax import lax
from jax.experimental import pallas as pl
from jax.experimental.pallas import tpu as pltpu
from jax.experimental.pallas import tpu_sc as plsc

_N = 10000
_D = 128
_E = 320000
_NC = 2
_NS = 16
_NH = _N // _NC
_NTRASH = 120
_NACC = _NH + _NTRASH
_RPT = _NACC // _NS
_ECHK = 128
_K = 160
_EPAD = _NS * _K * _ECHK
_NBUF = 2

_sc_mesh = plsc.VectorSubcoreMesh(
    core_axis_name="c", subcore_axis_name="s", num_cores=_NC, num_subcores=_NS
)


def _seg_sum_body(x_hbm, src_hbm, dst_hbm, zero_hbm, out_hbm,
                  src_v, dst_v, rows0, rows1, acc_sh,
                  zsem, gsem0, gsem1):
    c = lax.axis_index("c")
    s = lax.axis_index("s")
    rows = [rows0, rows1]
    gsem = [gsem0, gsem1]

    zcp = pltpu.async_copy(zero_hbm, acc_sh.at[pl.ds(s * _RPT, _RPT)], zsem)
    pltpu.sync_copy(src_hbm.at[s], src_v)
    pltpu.sync_copy(dst_hbm.at[c * _NS + s], dst_v)
    for b in range(_NBUF):
        pltpu.async_copy(x_hbm.at[src_v.at[b]], rows[b], gsem[b])
    zcp.wait()
    plsc.subcore_barrier()

    def group(g, carry):
        for b in range(_NBUF):
            j = g * _NBUF + b
            pltpu.make_async_copy(x_hbm.at[src_v.at[j]], rows[b], gsem[b]).wait()
            pltpu.sync_copy(rows[b], acc_sh.at[dst_v.at[j]], add=True)
            pltpu.async_copy(x_hbm.at[src_v.at[j + _NBUF]], rows[b], gsem[b])
        return carry

    lax.fori_loop(0, _K // _NBUF - 1, group, 0)

    for b in range(_NBUF):
        j = _K - _NBUF + b
        pltpu.make_async_copy(x_hbm.at[src_v.at[j]], rows[b], gsem[b]).wait()
        pltpu.sync_copy(rows[b], acc_sh.at[dst_v.at[j]], add=True)

    plsc.subcore_barrier()
    pltpu.sync_copy(acc_sh.at[pl.ds(s * _RPT, _RPT)],
                    out_hbm.at[c, pl.ds(s * _RPT, _RPT)])


_seg_sum = pl.kernel(
    _seg_sum_body,
    out_type=jax.ShapeDtypeStruct((_NC, _NACC, _D), jnp.float32),
    mesh=_sc_mesh,
    scratch_types=[
        pltpu.VMEM((_K, _ECHK), jnp.int32),
        pltpu.VMEM((_K, _ECHK), jnp.int32),
        pltpu.VMEM((_ECHK, _D), jnp.float32),
        pltpu.VMEM((_ECHK, _D), jnp.float32),
        pltpu.VMEM_SHARED((_NACC, _D), jnp.float32),
        pltpu.SemaphoreType.DMA,
        pltpu.SemaphoreType.DMA,
        pltpu.SemaphoreType.DMA,
    ],
)


_CH = 128
_NCHUNK = _EPAD // _CH
_ECHUNK = _E // _CH


def _edge_prep_body(edge_ref, src_out, dst_out):
    src2d = edge_ref[0].reshape(_ECHUNK, _CH)
    dst2d = edge_ref[1].reshape(_ECHUNK, _CH)
    npad = _NCHUNK - _ECHUNK
    padk = (lax.broadcasted_iota(jnp.int32, (npad, _CH), 0) * _CH
            + lax.broadcasted_iota(jnp.int32, (npad, _CH), 1))
    src_out[...] = jnp.concatenate(
        [src2d, (padk * 131) % _N], axis=0)
    eids = (lax.broadcasted_iota(jnp.int32, (_ECHUNK, _CH), 0) * _CH
            + lax.broadcasted_iota(jnp.int32, (_ECHUNK, _CH), 1))
    trash = _NH + (eids % _NTRASH)
    pad_trash = _NH + ((_E + padk) % _NTRASH)
    cores = []
    for cc in range(_NC):
        local = dst2d - cc * _NH
        own = (local >= 0) & (local < _NH)
        main = jnp.where(own, local, trash)
        cores.append(jnp.concatenate([main, pad_trash], axis=0))
    dst_out[...] = jnp.stack(cores)


_edge_prep = pl.pallas_call(
    _edge_prep_body,
    out_shape=(
        jax.ShapeDtypeStruct((_NCHUNK, _CH), jnp.int32),
        jax.ShapeDtypeStruct((_NC, _NCHUNK, _CH), jnp.int32),
    ),
)


_ROWS_BLK = 1000
_BPC = _NH // _ROWS_BLK


def _mlp1_body(x_ref, p_ref, w1_ref, b1_ref, w2_ref, b2_ref, o_ref):
    h = x_ref[...] + p_ref[0]
    a = jnp.dot(h, w1_ref[...], preferred_element_type=jnp.float32) + b1_ref[...]
    a = jnp.maximum(a, 0.0)
    z = jnp.dot(a, w2_ref[...], preferred_element_type=jnp.float32) + b2_ref[...]
    o_ref[...] = jnp.maximum(z, 0.0)


def _mlp2_body(h_ref, q_ref, w3_ref, b3_ref, w4_ref, b4_ref, o_ref):
    g = h_ref[...] + q_ref[0]
    a = jnp.dot(g, w3_ref[...], preferred_element_type=jnp.float32) + b3_ref[...]
    a = jnp.maximum(a, 0.0)
    z = jnp.dot(a, w4_ref[...], preferred_element_type=jnp.float32) + b4_ref[...]
    m = jnp.max(z, axis=1, keepdims=True)
    e = z - m
    o_ref[...] = e - jnp.log(jnp.sum(jnp.exp(e), axis=1, keepdims=True))


def _row_blocked_call(body):
    blk = lambda: pl.BlockSpec((_ROWS_BLK, _D), lambda i: (i, 0))
    part = pl.BlockSpec((1, _ROWS_BLK, _D), lambda i: (i // _BPC, i % _BPC, 0))
    full = lambda: pl.BlockSpec((_D, _D), lambda i: (0, 0))
    bias = lambda: pl.BlockSpec((1, _D), lambda i: (0, 0))
    return pl.pallas_call(
        body,
        grid=(_N // _ROWS_BLK,),
        in_specs=[blk(), part, full(), bias(), full(), bias()],
        out_specs=blk(),
        out_shape=jax.ShapeDtypeStruct((_N, _D), jnp.float32),
    )


_mlp1 = _row_blocked_call(_mlp1_body)
_mlp2 = _row_blocked_call(_mlp2_body)


def kernel(x, edge_index, W1, b1, W2, b2, W3, b3, W4, b4):
    srcp, dstp = _edge_prep(edge_index)
    srcp = srcp.reshape(_NS, _K, _ECHK)
    dst3d = dstp.reshape(_NC * _NS, _K, _ECHK)
    zeros = jnp.zeros((_RPT, _D), jnp.float32)

    p = _seg_sum(x, srcp, dst3d, zeros)
    h = _mlp1(x, p, W1, b1.reshape(1, _D), W2, b2.reshape(1, _D))
    q = _seg_sum(h, srcp, dst3d, zeros)
    return _mlp2(h, q, W3, b3.reshape(1, _D), W4, b4.reshape(1, _D))
